# baseline (speedup 1.0000x reference)
import functools
import math

import jax
import jax.numpy as jnp
from jax.experimental import pallas as pl
from jax.experimental.pallas import tpu as pltpu

B, S, D = 4, 512, 256
V = 32000
K = 32768
KSH = 64
BAND = 16
T = 4
BETA0, BETA1 = 1.0, 4.0
TAU0, TAU1 = 0.5, 0.1

N = B * S
TN = 128
VT = 3200
NT = N // TN
NV = V // VT
INV_SQRT_D = 1.0 / math.sqrt(D)


def _solver_readout_body(q_ref, msel_ref, xf_ref, an_ref, ant_ref, w_ref,
                         bias_ref, out_ref, y_scr):
    v = pl.program_id(1)

    @pl.when(v == 0)
    def _solve():
        q = q_ref[...]
        msel = msel_ref[...]
        xf = xf_ref[...]
        an = an_ref[...]
        ant = ant_ref[...]

        def dot_nk(yv):
            return jnp.sum(yv[:, None, :] * msel, axis=-1) * INV_SQRT_D

        def dot_nd(p):
            return jnp.sum(p[:, :, None] * msel, axis=1)

        def softmax(x):
            m = jnp.max(x, axis=-1, keepdims=True)
            e = jnp.exp(x - m)
            return e / jnp.sum(e, axis=-1, keepdims=True)

        s0 = dot_nk(q)
        p = softmax(s0)
        y = dot_nd(p)
        lam = jnp.zeros_like(y)
        for t in range(T):
            frac = t / (T - 1)
            beta = BETA0 + (BETA1 - BETA0) * frac
            tau = TAU0 + (TAU1 - TAU0) * frac
            sc = dot_nk(y)
            p = softmax(jnp.log(p + 1e-9) + beta * sc)
            yb = dot_nd(p)
            r = jax.lax.dot_general(y, an, (((1,), (0,)), ((), ())),
                                    preferred_element_type=jnp.float32) - xf
            lam = lam + tau * r
            y = y - tau * (jax.lax.dot_general(lam, ant, (((1,), (0,)), ((), ())),
                                               preferred_element_type=jnp.float32)
                           + (y - yb))
        y_scr[...] = y

    w = w_ref[...]
    out_ref[...] = jax.lax.dot_general(
        y_scr[...], w, (((1,), (1,)), ((), ())),
        preferred_element_type=jnp.float32) + bias_ref[...]


def _banded_norm_body(a_ref, an_ref):
    a = a_ref[...]
    row = jax.lax.broadcasted_iota(jnp.int32, (D, D), 0)
    col = jax.lax.broadcasted_iota(jnp.int32, (D, D), 1)
    band = jnp.abs(row - col) <= BAND
    a_b = jnp.where(band, a, 0.0)
    m = jax.lax.dot_general(a_b, a_b, (((0,), (0,)), ((), ())),
                            preferred_element_type=jnp.float32)
    mp = m / jnp.sqrt(jnp.sum(m * m))
    for _ in range(16):
        mp = jnp.dot(mp, mp, preferred_element_type=jnp.float32)
        mp = mp / jnp.sqrt(jnp.sum(mp * mp))
    u = 1.0 + 1e-3 * jax.lax.broadcasted_iota(jnp.int32, (1, D), 1).astype(jnp.float32)
    v = jnp.dot(u, mp, preferred_element_type=jnp.float32)
    t = jnp.dot(v, m, preferred_element_type=jnp.float32)
    sigma2 = jnp.sum(t * v) / jnp.sum(v * v)
    sigma = jnp.sqrt(sigma2)
    an_ref[...] = a_b / (sigma + 1e-6)


def _banded_norm(a):
    return pl.pallas_call(
        _banded_norm_body,
        out_shape=jax.ShapeDtypeStruct((D, D), jnp.float32),
    )(a)


def _solver_readout(q, m_sel, xf, a_n, a_nt, w_out, b_out):
    return pl.pallas_call(
        _solver_readout_body,
        grid=(NT, NV),
        in_specs=[
            pl.BlockSpec((TN, D), lambda t, v: (t, 0)),
            pl.BlockSpec((TN, KSH, D), lambda t, v: (t, 0, 0)),
            pl.BlockSpec((TN, D), lambda t, v: (t, 0)),
            pl.BlockSpec((D, D), lambda t, v: (0, 0)),
            pl.BlockSpec((D, D), lambda t, v: (0, 0)),
            pl.BlockSpec((VT, D), lambda t, v: (v, 0)),
            pl.BlockSpec((1, VT), lambda t, v: (0, v)),
        ],
        out_specs=pl.BlockSpec((TN, VT), lambda t, v: (t, v)),
        out_shape=jax.ShapeDtypeStruct((N, V), jnp.float32),
        scratch_shapes=[pltpu.VMEM((TN, D), jnp.float32)],
    )(q, m_sel, xf, a_n, a_nt, w_out, b_out)


def kernel(tokens, emb_table, memory, A, Ws1, Ws2, W_out, b_out):
    e = emb_table[tokens.reshape(-1)]
    eb = e.reshape(B, S, D)
    x = jnp.cumsum(eb, axis=1) / jnp.arange(1, S + 1, dtype=jnp.float32)[None, :, None]
    xf = x.reshape(N, D)

    scores = e @ memory.T
    _, kset = jax.lax.top_k(scores, KSH)
    m_sel = memory[kset]

    a_n = _banded_norm(A)

    q = jax.nn.relu(e @ Ws1) @ Ws2

    logits = _solver_readout(q, m_sel, xf, a_n, a_n.T, W_out,
                             b_out.reshape(1, V))
    return logits.reshape(B, S, V)


# speedup vs baseline: 5.1760x; 5.1760x over previous
import functools
import math

import jax
import jax.numpy as jnp
from jax import lax
from jax.experimental import pallas as pl
from jax.experimental.pallas import tpu as pltpu
from jax.experimental.pallas import tpu_sc as plsc

B, S, D = 4, 512, 256
V = 32000
K = 32768
KSH = 64
BAND = 16
T = 4
BETA0, BETA1 = 1.0, 4.0
TAU0, TAU1 = 0.5, 0.1

N = B * S
TN = 128
VT = 3200
NT = N // TN
NV = V // VT
INV_SQRT_D = 1.0 / math.sqrt(D)


def _solver_readout_body(q_ref, msel_ref, xf_ref, an_ref, ant_ref, w_ref,
                         bias_ref, out_ref, y_scr):
    v = pl.program_id(1)

    @pl.when(v == 0)
    def _solve():
        q = q_ref[...]
        msel = msel_ref[...]
        xf = xf_ref[...]
        an = an_ref[...]
        ant = ant_ref[...]

        def dot_nk(yv):
            return jnp.sum(yv[:, None, :] * msel, axis=-1) * INV_SQRT_D

        def dot_nd(p):
            return jnp.sum(p[:, :, None] * msel, axis=1)

        def softmax(x):
            m = jnp.max(x, axis=-1, keepdims=True)
            e = jnp.exp(x - m)
            return e / jnp.sum(e, axis=-1, keepdims=True)

        s0 = dot_nk(q)
        p = softmax(s0)
        y = dot_nd(p)
        lam = jnp.zeros_like(y)
        for t in range(T):
            frac = t / (T - 1)
            beta = BETA0 + (BETA1 - BETA0) * frac
            tau = TAU0 + (TAU1 - TAU0) * frac
            sc = dot_nk(y)
            p = softmax(jnp.log(p + 1e-9) + beta * sc)
            yb = dot_nd(p)
            r = jax.lax.dot_general(y, an, (((1,), (0,)), ((), ())),
                                    preferred_element_type=jnp.float32) - xf
            lam = lam + tau * r
            y = y - tau * (jax.lax.dot_general(lam, ant, (((1,), (0,)), ((), ())),
                                               preferred_element_type=jnp.float32)
                           + (y - yb))
        y_scr[...] = y

    w = w_ref[...]
    out_ref[...] = jax.lax.dot_general(
        y_scr[...], w, (((1,), (1,)), ((), ())),
        preferred_element_type=jnp.float32) + bias_ref[...]


_SC_INFO = plsc.get_sparse_core_info()
_NC, _NS = _SC_INFO.num_cores, _SC_INFO.num_subcores
NW = _NC * _NS


def _emb_gather(emb_table, tokens_flat):
    rows_per_w = N // NW
    mesh = plsc.VectorSubcoreMesh(core_axis_name="c", subcore_axis_name="s")

    @functools.partial(
        pl.kernel, mesh=mesh,
        compiler_params=pltpu.CompilerParams(needs_layout_passes=False),
        out_type=jax.ShapeDtypeStruct((N, D), jnp.float32),
        scratch_types=[
            pltpu.VMEM((rows_per_w,), jnp.int32),
            pltpu.VMEM((rows_per_w, D), jnp.float32),
            pltpu.SemaphoreType.DMA,
        ],
    )
    def k(table_hbm, idx_hbm, out_hbm, idx_v, rows_v, sem):
        wid = lax.axis_index("s") * _NC + lax.axis_index("c")
        base = wid * rows_per_w
        pltpu.sync_copy(idx_hbm.at[pl.ds(base, rows_per_w)], idx_v)
        pltpu.async_copy(table_hbm.at[idx_v], rows_v, sem).wait()
        pltpu.sync_copy(rows_v, out_hbm.at[pl.ds(base, rows_per_w)])

    return k(emb_table, tokens_flat)


CH = 16
NCH = K // CH
CAP = 80
CAPS = 1024
ROWS_W = N // NW
TM = 256
MB = 4096


def _scores_body(e_ref, mem_ref, s_ref, cmx_ref):
    s = jax.lax.dot_general(e_ref[...], mem_ref[...], (((1,), (1,)), ((), ())),
                            preferred_element_type=jnp.float32)
    s_ref[...] = s
    cmx_ref[...] = jnp.max(s.reshape(TM, MB // CH, CH), axis=-1)


def _scores_chunkmax(e, memory):
    return pl.pallas_call(
        _scores_body,
        grid=(N // TM, K // MB),
        in_specs=[
            pl.BlockSpec((TM, D), lambda t, m: (t, 0)),
            pl.BlockSpec((MB, D), lambda t, m: (m, 0)),
        ],
        out_specs=[
            pl.BlockSpec((TM, MB), lambda t, m: (t, m)),
            pl.BlockSpec((TM, MB // CH), lambda t, m: (t, m)),
        ],
        out_shape=[
            jax.ShapeDtypeStruct((N, K), jnp.float32),
            jax.ShapeDtypeStruct((N, NCH), jnp.float32),
        ],
    )(e, memory)


def _f2key(x):
    xi = jax.lax.bitcast_convert_type(x, jnp.int32)
    return xi ^ (jax.lax.shift_right_arithmetic(xi, 31) & jnp.int32(0x7FFFFFFF))


RG = 256


def _chunk_thresh_body(cmx_ref, tc_ref):
    keys = _f2key(cmx_ref[...])
    lo = jnp.min(keys, axis=-1, keepdims=True)
    hi = jnp.max(keys, axis=-1, keepdims=True) + 1
    for _ in range(24):
        mid = lo + jax.lax.shift_right_logical(hi - lo, 1)
        cnt = jnp.sum((keys >= mid).astype(jnp.int32), axis=-1, keepdims=True)
        ge = cnt >= KSH
        lo = jnp.where(ge, mid, lo)
        hi = jnp.where(ge, hi, mid)
    tc_ref[...] = jnp.broadcast_to(lo, (RG, 128))


def _chunk_thresh(cmx):
    out = pl.pallas_call(
        _chunk_thresh_body,
        grid=(N // RG,),
        in_specs=[pl.BlockSpec((RG, NCH), lambda g: (g, 0))],
        out_specs=pl.BlockSpec((RG, 128), lambda g: (g, 0)),
        out_shape=jax.ShapeDtypeStruct((N, 128), jnp.int32),
    )(cmx)
    return out[:, 0]


def _key2f(k):
    return jax.lax.bitcast_convert_type(
        k ^ (jax.lax.shift_right_arithmetic(k, 31) & jnp.int32(0x7FFFFFFF)),
        jnp.float32)


def _popcnt_scalar(mask):
    return jnp.max(plsc.all_reduce_population_count(mask))


def _topk_gather(cmx, tc, scores128, memory):
    mesh = plsc.VectorSubcoreMesh(core_axis_name="c", subcore_axis_name="s")

    @functools.partial(
        pl.kernel, mesh=mesh,
        compiler_params=pltpu.CompilerParams(needs_layout_passes=False),
        out_type=jax.ShapeDtypeStruct((N * KSH, D), jnp.float32),
        scratch_types=[
            pltpu.VMEM((1, NCH), jnp.float32),
            pltpu.VMEM((ROWS_W,), jnp.int32),
            pltpu.VMEM((CAP + 16,), jnp.int32),
            pltpu.VMEM((CAP,), jnp.int32),
            pltpu.VMEM((CAP, 128), jnp.float32),
            pltpu.VMEM((CAPS + 16,), jnp.int32),
            pltpu.VMEM((CAPS + 16,), jnp.int32),
            pltpu.VMEM((80,), jnp.int32),
            pltpu.VMEM((KSH,), jnp.int32),
            pltpu.VMEM((KSH, D), jnp.float32),
            pltpu.SemaphoreType.DMA,
            pltpu.SemaphoreType.DMA,
        ],
    )
    def k3(cm_hbm, tc_hbm, s128_hbm, mem_hbm, out_hbm,
           cm_v, tc_v, cidx_v, sidx_v, cand_v, surv_k, surv_g,
           sel80, sel64, msel_v, sem1, sem2):
        wid = lax.axis_index("s") * _NC + lax.axis_index("c")
        base_r = wid * ROWS_W
        pltpu.sync_copy(tc_hbm.at[pl.ds(base_r, ROWS_W)], tc_v)
        zero16 = jnp.zeros((16,), jnp.int32)
        for jj in range(CAP // 16):
            cidx_v[pl.ds(jj * 16, 16)] = zero16

        def row_body(i, _):
            r = base_r + i
            pltpu.sync_copy(cm_hbm.at[pl.ds(r, 1)], cm_v)
            tvr = tc_v[pl.ds((i // 16) * 16, 16)]
            lane = jax.lax.broadcasted_iota(jnp.int32, (16,), 0) == (i % 16)
            tck_s = jnp.sum(jnp.where(lane, tvr, 0))
            tck = jnp.full((16,), tck_s, jnp.int32)
            tcf1 = _key2f(tck)

            ptr = jnp.int32(0)
            for j in range(NCH // 16):
                v = cm_v[0, pl.ds(j * 16, 16)]
                m = v >= tcf1
                ci = jax.lax.broadcasted_iota(jnp.int32, (16,), 0) + j * 16
                plsc.store_compressed(
                    cidx_v.at[pl.ds(jnp.minimum(ptr, CAP), 16)], ci, mask=m)
                ptr = ptr + _popcnt_scalar(m)
            n_ch = jnp.minimum(ptr, CAP)

            for jj in range(CAP // 16):
                cv = cidx_v[pl.ds(jj * 16, 16)]
                sidx_v[pl.ds(jj * 16, 16)] = (
                    jax.lax.shift_right_logical(cv, 3) + r * (K // 128))
            pltpu.async_copy(s128_hbm.at[sidx_v], cand_v, sem1).wait()

            iota16 = jax.lax.broadcasted_iota(jnp.int32, (16,), 0)

            def pf_body(j, sp):
                cvr = cidx_v[pl.ds((j // 16) * 16, 16)]
                cvj = jnp.sum(jnp.where(iota16 == (j % 16), cvr, 0))
                off = (cvj & 7) * 16
                vals = cand_v[j, pl.ds(off, 16)]
                gvx = cvj * 16 + iota16
                m = vals >= tcf1
                spc = jnp.minimum(sp, CAPS)
                kv = _f2key(vals)
                plsc.store_compressed(surv_k.at[pl.ds(spc, 16)], kv, mask=m)
                plsc.store_compressed(surv_g.at[pl.ds(spc, 16)], gvx, mask=m)
                return sp + _popcnt_scalar(m)

            sp = jax.lax.fori_loop(0, n_ch, pf_body, jnp.int32(0))
            sp = jnp.minimum(sp, CAPS)
            surv_k[pl.ds(sp, 16)] = jnp.full((16,), jnp.int32(-0x80000000))
            surv_g[pl.ds(sp, 16)] = zero16
            nsv = (sp + 15) // 16

            def count_ge(t):
                def cb(j, acc):
                    kv = surv_k[pl.ds(j * 16, 16)]
                    return acc + plsc.all_reduce_population_count(kv >= t)
                return jnp.max(jax.lax.fori_loop(0, nsv, cb,
                                                 jnp.zeros((16,), jnp.int32)))

            def wcond(c):
                lo, hi = c
                return hi - lo > 1

            def wbody(c):
                lo, hi = c
                mid = lo + jax.lax.shift_right_logical(hi - lo, 1)
                cnt = count_ge(mid)
                ge = cnt >= KSH
                return (jnp.where(ge, mid, lo), jnp.where(ge, hi, mid))

            lo0 = tck_s
            hi0 = jnp.int32(0x7F800001)
            lo, hi = jax.lax.while_loop(wcond, wbody, (lo0, hi0))
            c_gt = count_ge(hi)
            eq_budget = KSH - c_gt

            def sel_body(j, c):
                nsel, eq_taken = c
                kv = surv_k[pl.ds(j * 16, 16)]
                gv = surv_g[pl.ds(j * 16, 16)]
                m_gt = kv >= hi
                m_eq = kv == lo
                eq_rank = plsc.cumsum(jnp.where(m_eq, 1, 0))
                take_eq = m_eq & ((eq_taken + eq_rank) <= eq_budget)
                msel = m_gt | take_eq
                plsc.store_compressed(sel80.at[pl.ds(nsel, 16)], gv, mask=msel)
                return (nsel + _popcnt_scalar(msel),
                        eq_taken + _popcnt_scalar(take_eq))

            jax.lax.fori_loop(0, nsv, sel_body, (jnp.int32(0), jnp.int32(0)))
            for jj in range(KSH // 16):
                sel64[pl.ds(jj * 16, 16)] = sel80[pl.ds(jj * 16, 16)]

            pltpu.async_copy(mem_hbm.at[sel64], msel_v, sem1).wait()
            pltpu.sync_copy(msel_v, out_hbm.at[pl.ds(r * KSH, KSH)])
            return 0

        jax.lax.fori_loop(0, ROWS_W, row_body, 0)

    return k3(cmx, tc, scores128, memory)


def _banded_norm_body(a_ref, an_ref):
    a = a_ref[...]
    row = jax.lax.broadcasted_iota(jnp.int32, (D, D), 0)
    col = jax.lax.broadcasted_iota(jnp.int32, (D, D), 1)
    band = jnp.abs(row - col) <= BAND
    a_b = jnp.where(band, a, 0.0)
    m = jax.lax.dot_general(a_b, a_b, (((0,), (0,)), ((), ())),
                            preferred_element_type=jnp.float32)
    mp = m / jnp.sqrt(jnp.sum(m * m))
    for _ in range(16):
        mp = jnp.dot(mp, mp, preferred_element_type=jnp.float32)
        mp = mp / jnp.sqrt(jnp.sum(mp * mp))
    u = 1.0 + 1e-3 * jax.lax.broadcasted_iota(jnp.int32, (1, D), 1).astype(jnp.float32)
    v = jnp.dot(u, mp, preferred_element_type=jnp.float32)
    t = jnp.dot(v, m, preferred_element_type=jnp.float32)
    sigma2 = jnp.sum(t * v) / jnp.sum(v * v)
    sigma = jnp.sqrt(sigma2)
    an_ref[...] = a_b / (sigma + 1e-6)


def _banded_norm(a):
    return pl.pallas_call(
        _banded_norm_body,
        out_shape=jax.ShapeDtypeStruct((D, D), jnp.float32),
    )(a)


def _solver_readout(q, m_sel, xf, a_n, a_nt, w_out, b_out):
    return pl.pallas_call(
        _solver_readout_body,
        grid=(NT, NV),
        in_specs=[
            pl.BlockSpec((TN, D), lambda t, v: (t, 0)),
            pl.BlockSpec((TN, KSH, D), lambda t, v: (t, 0, 0)),
            pl.BlockSpec((TN, D), lambda t, v: (t, 0)),
            pl.BlockSpec((D, D), lambda t, v: (0, 0)),
            pl.BlockSpec((D, D), lambda t, v: (0, 0)),
            pl.BlockSpec((VT, D), lambda t, v: (v, 0)),
            pl.BlockSpec((1, VT), lambda t, v: (0, v)),
        ],
        out_specs=pl.BlockSpec((TN, VT), lambda t, v: (t, v)),
        out_shape=jax.ShapeDtypeStruct((N, V), jnp.float32),
        scratch_shapes=[pltpu.VMEM((TN, D), jnp.float32)],
    )(q, m_sel, xf, a_n, a_nt, w_out, b_out)


def kernel(tokens, emb_table, memory, A, Ws1, Ws2, W_out, b_out):
    e = _emb_gather(emb_table, tokens.reshape(-1).astype(jnp.int32))
    eb = e.reshape(B, S, D)
    x = jnp.cumsum(eb, axis=1) / jnp.arange(1, S + 1, dtype=jnp.float32)[None, :, None]
    xf = x.reshape(N, D)

    scores, cmx = _scores_chunkmax(e, memory)
    tc = _chunk_thresh(cmx)
    m_sel = _topk_gather(cmx, tc, scores.reshape(N * K // 128, 128),
                         memory).reshape(N, KSH, D)

    a_n = _banded_norm(A)

    q = jax.nn.relu(e @ Ws1) @ Ws2

    logits = _solver_readout(q, m_sel, xf, a_n, a_n.T, W_out,
                             b_out.reshape(1, V))
    return logits.reshape(B, S, V)


# speedup vs baseline: 5.9591x; 1.1513x over previous
import functools
import math

import jax
import jax.numpy as jnp
from jax import lax
from jax.experimental import pallas as pl
from jax.experimental.pallas import tpu as pltpu
from jax.experimental.pallas import tpu_sc as plsc

B, S, D = 4, 512, 256
V = 32000
K = 32768
KSH = 64
BAND = 16
T = 4
BETA0, BETA1 = 1.0, 4.0
TAU0, TAU1 = 0.5, 0.1

N = B * S
TN = 128
VT = 3200
NT = N // TN
NV = V // VT
INV_SQRT_D = 1.0 / math.sqrt(D)


def _solver_readout_body(q_ref, msel_ref, xf_ref, an_ref, ant_ref, w_ref,
                         bias_ref, out_ref, y_scr):
    v = pl.program_id(1)

    @pl.when(v == 0)
    def _solve():
        q = q_ref[...]
        msel = msel_ref[...]
        xf = xf_ref[...]
        an = an_ref[...]
        ant = ant_ref[...]

        def dot_nk(yv):
            return jnp.sum(yv[:, None, :] * msel, axis=-1) * INV_SQRT_D

        def dot_nd(p):
            return jnp.sum(p[:, :, None] * msel, axis=1)

        def softmax(x):
            m = jnp.max(x, axis=-1, keepdims=True)
            e = jnp.exp(x - m)
            return e / jnp.sum(e, axis=-1, keepdims=True)

        s0 = dot_nk(q)
        p = softmax(s0)
        y = dot_nd(p)
        lam = jnp.zeros_like(y)
        for t in range(T):
            frac = t / (T - 1)
            beta = BETA0 + (BETA1 - BETA0) * frac
            tau = TAU0 + (TAU1 - TAU0) * frac
            sc = dot_nk(y)
            p = softmax(jnp.log(p + 1e-9) + beta * sc)
            yb = dot_nd(p)
            r = jax.lax.dot_general(y, an, (((1,), (0,)), ((), ())),
                                    preferred_element_type=jnp.float32) - xf
            lam = lam + tau * r
            y = y - tau * (jax.lax.dot_general(lam, ant, (((1,), (0,)), ((), ())),
                                               preferred_element_type=jnp.float32)
                           + (y - yb))
        y_scr[...] = y

    w = w_ref[...]
    out_ref[...] = jax.lax.dot_general(
        y_scr[...], w, (((1,), (1,)), ((), ())),
        preferred_element_type=jnp.float32) + bias_ref[...]


_SC_INFO = plsc.get_sparse_core_info()
_NC, _NS = _SC_INFO.num_cores, _SC_INFO.num_subcores
NW = _NC * _NS


def _emb_gather(emb_table, tokens_flat):
    rows_per_w = N // NW
    mesh = plsc.VectorSubcoreMesh(core_axis_name="c", subcore_axis_name="s")

    @functools.partial(
        pl.kernel, mesh=mesh,
        compiler_params=pltpu.CompilerParams(needs_layout_passes=False),
        out_type=jax.ShapeDtypeStruct((N, D), jnp.float32),
        scratch_types=[
            pltpu.VMEM((rows_per_w,), jnp.int32),
            pltpu.VMEM((rows_per_w, D), jnp.float32),
            pltpu.SemaphoreType.DMA,
        ],
    )
    def k(table_hbm, idx_hbm, out_hbm, idx_v, rows_v, sem):
        wid = lax.axis_index("s") * _NC + lax.axis_index("c")
        base = wid * rows_per_w
        pltpu.sync_copy(idx_hbm.at[pl.ds(base, rows_per_w)], idx_v)
        pltpu.async_copy(table_hbm.at[idx_v], rows_v, sem).wait()
        pltpu.sync_copy(rows_v, out_hbm.at[pl.ds(base, rows_per_w)])

    return k(emb_table, tokens_flat)


CH = 16
NCH = K // CH
CAP = 80
CAPS = 1024
ROWS_W = N // NW
TM = 256
MB = 4096


def _scores_body(e_ref, mem_ref, s_ref, cmx_ref):
    s = jax.lax.dot_general(e_ref[...], mem_ref[...], (((1,), (1,)), ((), ())),
                            preferred_element_type=jnp.float32)
    s_ref[...] = s
    cmx_ref[...] = jnp.max(s.reshape(TM, MB // CH, CH), axis=-1)


def _scores_chunkmax(e, memory):
    return pl.pallas_call(
        _scores_body,
        grid=(N // TM, K // MB),
        in_specs=[
            pl.BlockSpec((TM, D), lambda t, m: (t, 0)),
            pl.BlockSpec((MB, D), lambda t, m: (m, 0)),
        ],
        out_specs=[
            pl.BlockSpec((TM, MB), lambda t, m: (t, m)),
            pl.BlockSpec((TM, MB // CH), lambda t, m: (t, m)),
        ],
        out_shape=[
            jax.ShapeDtypeStruct((N, K), jnp.float32),
            jax.ShapeDtypeStruct((N, NCH), jnp.float32),
        ],
    )(e, memory)


def _f2key(x):
    xi = jax.lax.bitcast_convert_type(x, jnp.int32)
    return xi ^ (jax.lax.shift_right_arithmetic(xi, 31) & jnp.int32(0x7FFFFFFF))


RG = 256


def _chunk_thresh_body(cmx_ref, tc_ref):
    keys = _f2key(cmx_ref[...])
    lo = jnp.min(keys, axis=-1, keepdims=True)
    hi = jnp.max(keys, axis=-1, keepdims=True) + 1
    for _ in range(24):
        mid = lo + jax.lax.shift_right_logical(hi - lo, 1)
        cnt = jnp.sum((keys >= mid).astype(jnp.int32), axis=-1, keepdims=True)
        ge = cnt >= KSH
        lo = jnp.where(ge, mid, lo)
        hi = jnp.where(ge, hi, mid)
    tc_ref[...] = jnp.broadcast_to(lo, (RG, 128))


def _chunk_thresh(cmx):
    out = pl.pallas_call(
        _chunk_thresh_body,
        grid=(N // RG,),
        in_specs=[pl.BlockSpec((RG, NCH), lambda g: (g, 0))],
        out_specs=pl.BlockSpec((RG, 128), lambda g: (g, 0)),
        out_shape=jax.ShapeDtypeStruct((N, 128), jnp.int32),
    )(cmx)
    return out[:, 0]


def _key2f(k):
    return jax.lax.bitcast_convert_type(
        k ^ (jax.lax.shift_right_arithmetic(k, 31) & jnp.int32(0x7FFFFFFF)),
        jnp.float32)


def _popcnt_scalar(mask):
    return jnp.max(plsc.all_reduce_population_count(mask))


def _topk_gather(cmx, tc, scores, memory):
    mesh = plsc.VectorSubcoreMesh(core_axis_name="c", subcore_axis_name="s")

    @functools.partial(
        pl.kernel, mesh=mesh,
        compiler_params=pltpu.CompilerParams(needs_layout_passes=False),
        out_type=jax.ShapeDtypeStruct((N * KSH, D), jnp.float32),
        scratch_types=[
            pltpu.VMEM((2, NCH), jnp.float32),
            pltpu.VMEM((2, K), jnp.float32),
            pltpu.VMEM((ROWS_W,), jnp.int32),
            pltpu.VMEM((CAP + 16,), jnp.int32),
            pltpu.VMEM((CAPS + 16,), jnp.int32),
            pltpu.VMEM((CAPS + 16,), jnp.int32),
            pltpu.VMEM((80,), jnp.int32),
            pltpu.VMEM((KSH,), jnp.int32),
            pltpu.VMEM((KSH, D), jnp.float32),
            pltpu.SemaphoreType.DMA,
            pltpu.SemaphoreType.DMA,
            pltpu.SemaphoreType.DMA,
        ],
    )
    def k3(cm_hbm, tc_hbm, s_hbm, mem_hbm, out_hbm,
           cm_v, row_v, tc_v, cidx_v, surv_k, surv_g,
           sel80, sel64, msel_v, sem0, sem1, semg):
        wid = lax.axis_index("s") * _NC + lax.axis_index("c")
        base_r = wid * ROWS_W
        pltpu.sync_copy(tc_hbm.at[pl.ds(base_r, ROWS_W)], tc_v)
        zero16 = jnp.zeros((16,), jnp.int32)
        iota16 = jax.lax.broadcasted_iota(jnp.int32, (16,), 0)
        for jj in range(CAP // 16):
            cidx_v[pl.ds(jj * 16, 16)] = zero16

        sems = (sem0, sem1)
        pltpu.async_copy(cm_hbm.at[pl.ds(base_r, 1)], cm_v.at[pl.ds(0, 1)],
                         sem0)
        pltpu.async_copy(s_hbm.at[pl.ds(base_r, 1)], row_v.at[pl.ds(0, 1)],
                         sem0)

        def process_row(i, par):
            r = base_r + i
            nxt = base_r + jnp.minimum(i + 1, ROWS_W - 1)
            op = 1 - par
            pltpu.async_copy(cm_hbm.at[pl.ds(nxt, 1)],
                             cm_v.at[pl.ds(op, 1)], sems[op])
            pltpu.async_copy(s_hbm.at[pl.ds(nxt, 1)],
                             row_v.at[pl.ds(op, 1)], sems[op])
            pltpu.make_async_copy(cm_hbm.at[pl.ds(r, 1)],
                                  cm_v.at[pl.ds(par, 1)], sems[par]).wait()
            pltpu.make_async_copy(s_hbm.at[pl.ds(r, 1)],
                                  row_v.at[pl.ds(par, 1)], sems[par]).wait()

            tvr = tc_v[pl.ds((i // 16) * 16, 16)]
            lane = iota16 == (i % 16)
            tck_s = jnp.sum(jnp.where(lane, tvr, 0))
            tck = jnp.full((16,), tck_s, jnp.int32)
            tcf1 = _key2f(tck)

            ptr = jnp.int32(0)
            for j in range(NCH // 16):
                v = cm_v[par, pl.ds(j * 16, 16)]
                m = v >= tcf1
                ci = iota16 + j * 16
                plsc.store_compressed(
                    cidx_v.at[pl.ds(jnp.minimum(ptr, CAP), 16)], ci, mask=m)
                ptr = ptr + _popcnt_scalar(m)
            n_ch = jnp.minimum(ptr, CAP)

            def pf_body(j, sp):
                cvr = cidx_v[pl.ds((j // 16) * 16, 16)]
                cvj = jnp.sum(jnp.where(iota16 == (j % 16), cvr, 0))
                vals = row_v[par, pl.ds(cvj * CH, 16)]
                gvx = cvj * CH + iota16
                m = vals >= tcf1
                spc = jnp.minimum(sp, CAPS)
                kv = _f2key(vals)
                plsc.store_compressed(surv_k.at[pl.ds(spc, 16)], kv, mask=m)
                plsc.store_compressed(surv_g.at[pl.ds(spc, 16)], gvx, mask=m)
                return sp + _popcnt_scalar(m)

            sp = jax.lax.fori_loop(0, n_ch, pf_body, jnp.int32(0))
            sp = jnp.minimum(sp, CAPS)
            surv_k[pl.ds(sp, 16)] = jnp.full((16,), jnp.int32(-0x80000000))
            surv_g[pl.ds(sp, 16)] = zero16
            nsv = (sp + 15) // 16

            def count_ge(t):
                def cb(j, acc):
                    kv = surv_k[pl.ds(j * 16, 16)]
                    return acc + plsc.all_reduce_population_count(kv >= t)
                return jnp.max(jax.lax.fori_loop(0, nsv, cb,
                                                 jnp.zeros((16,), jnp.int32)))

            def wcond(c):
                lo, hi = c
                return hi - lo > 1

            def wbody(c):
                lo, hi = c
                mid = lo + jax.lax.shift_right_logical(hi - lo, 1)
                cnt = count_ge(mid)
                ge = cnt >= KSH
                return (jnp.where(ge, mid, lo), jnp.where(ge, hi, mid))

            lo0 = tck_s
            hi0 = jnp.int32(0x7F800001)
            lo, hi = jax.lax.while_loop(wcond, wbody, (lo0, hi0))
            c_gt = count_ge(hi)
            eq_budget = KSH - c_gt

            def sel_body(j, c):
                nsel, eq_taken = c
                kv = surv_k[pl.ds(j * 16, 16)]
                gv = surv_g[pl.ds(j * 16, 16)]
                m_gt = kv >= hi
                m_eq = kv == lo
                eq_rank = plsc.cumsum(jnp.where(m_eq, 1, 0))
                take_eq = m_eq & ((eq_taken + eq_rank) <= eq_budget)
                msel = m_gt | take_eq
                plsc.store_compressed(sel80.at[pl.ds(nsel, 16)], gv, mask=msel)
                return (nsel + _popcnt_scalar(msel),
                        eq_taken + _popcnt_scalar(take_eq))

            jax.lax.fori_loop(0, nsv, sel_body, (jnp.int32(0), jnp.int32(0)))
            for jj in range(KSH // 16):
                sel64[pl.ds(jj * 16, 16)] = sel80[pl.ds(jj * 16, 16)]

            pltpu.async_copy(mem_hbm.at[sel64], msel_v, semg).wait()
            pltpu.sync_copy(msel_v, out_hbm.at[pl.ds(r * KSH, KSH)])

        def pair_body(p, _):
            process_row(p * 2, 0)
            process_row(p * 2 + 1, 1)
            return 0

        jax.lax.fori_loop(0, ROWS_W // 2, pair_body, 0)
        pltpu.make_async_copy(cm_hbm.at[pl.ds(base_r, 1)],
                              cm_v.at[pl.ds(0, 1)], sem0).wait()
        pltpu.make_async_copy(s_hbm.at[pl.ds(base_r, 1)],
                              row_v.at[pl.ds(0, 1)], sem0).wait()

    return k3(cmx, tc, scores, memory)


def _banded_norm_body(a_ref, an_ref):
    a = a_ref[...]
    row = jax.lax.broadcasted_iota(jnp.int32, (D, D), 0)
    col = jax.lax.broadcasted_iota(jnp.int32, (D, D), 1)
    band = jnp.abs(row - col) <= BAND
    a_b = jnp.where(band, a, 0.0)
    m = jax.lax.dot_general(a_b, a_b, (((0,), (0,)), ((), ())),
                            preferred_element_type=jnp.float32)
    mp = m / jnp.sqrt(jnp.sum(m * m))
    for _ in range(16):
        mp = jnp.dot(mp, mp, preferred_element_type=jnp.float32)
        mp = mp / jnp.sqrt(jnp.sum(mp * mp))
    u = 1.0 + 1e-3 * jax.lax.broadcasted_iota(jnp.int32, (1, D), 1).astype(jnp.float32)
    v = jnp.dot(u, mp, preferred_element_type=jnp.float32)
    t = jnp.dot(v, m, preferred_element_type=jnp.float32)
    sigma2 = jnp.sum(t * v) / jnp.sum(v * v)
    sigma = jnp.sqrt(sigma2)
    an_ref[...] = a_b / (sigma + 1e-6)


def _banded_norm(a):
    return pl.pallas_call(
        _banded_norm_body,
        out_shape=jax.ShapeDtypeStruct((D, D), jnp.float32),
    )(a)


def _solver_readout(q, m_sel, xf, a_n, a_nt, w_out, b_out):
    return pl.pallas_call(
        _solver_readout_body,
        grid=(NT, NV),
        in_specs=[
            pl.BlockSpec((TN, D), lambda t, v: (t, 0)),
            pl.BlockSpec((TN, KSH, D), lambda t, v: (t, 0, 0)),
            pl.BlockSpec((TN, D), lambda t, v: (t, 0)),
            pl.BlockSpec((D, D), lambda t, v: (0, 0)),
            pl.BlockSpec((D, D), lambda t, v: (0, 0)),
            pl.BlockSpec((VT, D), lambda t, v: (v, 0)),
            pl.BlockSpec((1, VT), lambda t, v: (0, v)),
        ],
        out_specs=pl.BlockSpec((TN, VT), lambda t, v: (t, v)),
        out_shape=jax.ShapeDtypeStruct((N, V), jnp.float32),
        scratch_shapes=[pltpu.VMEM((TN, D), jnp.float32)],
    )(q, m_sel, xf, a_n, a_nt, w_out, b_out)


def kernel(tokens, emb_table, memory, A, Ws1, Ws2, W_out, b_out):
    e = _emb_gather(emb_table, tokens.reshape(-1).astype(jnp.int32))
    eb = e.reshape(B, S, D)
    x = jnp.cumsum(eb, axis=1) / jnp.arange(1, S + 1, dtype=jnp.float32)[None, :, None]
    xf = x.reshape(N, D)

    scores, cmx = _scores_chunkmax(e, memory)
    tc = _chunk_thresh(cmx)
    m_sel = _topk_gather(cmx, tc, scores, memory).reshape(N, KSH, D)

    a_n = _banded_norm(A)

    q = jax.nn.relu(e @ Ws1) @ Ws2

    logits = _solver_readout(q, m_sel, xf, a_n, a_n.T, W_out,
                             b_out.reshape(1, V))
    return logits.reshape(B, S, V)
